# baseline (speedup 1.0000x reference)
import functools

import jax
import jax.numpy as jnp
from jax import lax
from jax.experimental import pallas as pl
from jax.experimental.pallas import tpu as pltpu
from jax.experimental.pallas import tpu_sc as plsc

_D = 32
_NC = 2
_NS = 16
_NW = _NC * _NS
_BL = 128
_NB = 4


@functools.lru_cache(maxsize=None)
def _make_gather(T: int):
    tri, tin = T // 8, 8
    mesh = plsc.VectorSubcoreMesh(core_axis_name="c", subcore_axis_name="s")

    @functools.partial(
        pl.kernel,
        mesh=mesh,
        out_type=jax.ShapeDtypeStruct((T, _D // 8, _NW, 8, _BL), jnp.float32),
        scratch_types=[
            pltpu.VMEM((tri, tin, _BL), jnp.int32),
            pltpu.VMEM((_NB, _BL, _D), jnp.float32),
            pltpu.VMEM((_NB, _D // 8, 8, _BL), jnp.float32),
            pltpu.SemaphoreType.DMA((_NB,)),
            pltpu.SemaphoreType.DMA((_NB,)),
        ],
        compiler_params=pltpu.CompilerParams(
            use_tc_tiling_on_sc=False, needs_layout_passes=False
        ),
    )
    def gather(idx5_hbm, table_hbm, out5_hbm, idx_v, rows_v, outt_v, gsem, ssem):
        wid = lax.axis_index("s") * _NC + lax.axis_index("c")
        pltpu.sync_copy(idx5_hbm.at[:, wid], idx_v)

        iota16 = lax.iota(jnp.int32, 16)

        def g_copy(t, b):
            return pltpu.make_async_copy(
                table_hbm.at[idx_v.at[t // 8, t % 8]], rows_v.at[b], gsem.at[b]
            )

        def s_copy(t, b):
            return pltpu.make_async_copy(
                outt_v.at[b], out5_hbm.at[t, :, wid], ssem.at[b]
            )

        for b in range(_NB):
            g_copy(b, b).start()

        def body(to, carry):
            for b in range(_NB):
                t = to * _NB + b
                g_copy(t, b).wait()

                @pl.when(t >= _NB)
                def _():
                    s_copy(t - _NB, b).wait()

                for cr in range(_D // 8):
                    for ci in range(8):
                        cvec = jnp.full((16,), 8 * cr + ci, jnp.int32)
                        for k in range(8):
                            v = plsc.load_gather(
                                rows_v.at[b], [iota16 + 16 * k, cvec]
                            )
                            outt_v[b, cr, ci, pl.ds(16 * k, 16)] = v

                s_copy(t, b).start()

                @pl.when(t + _NB < T)
                def _():
                    g_copy(t + _NB, b).start()

            return carry

        lax.fori_loop(0, T // _NB, body, 0)

        for b in range(_NB):
            s_copy(T - _NB + b, b).wait()

    return gather


def kernel(indices, W):
    rows, cols = indices.shape
    idx5 = indices.reshape(_NW, _BL, cols // 8, 8).transpose(2, 0, 3, 1)
    out5 = _make_gather(cols)(idx5, W)
    return out5.transpose(2, 4, 0, 1, 3).reshape(rows, cols, _D)


# speedup vs baseline: 1.5743x; 1.5743x over previous
import functools

import jax
import jax.numpy as jnp
from jax import lax
from jax.experimental import pallas as pl
from jax.experimental.pallas import tpu as pltpu
from jax.experimental.pallas import tpu_sc as plsc

_D = 32
_NC = 2
_NS = 16
_NW = _NC * _NS
_BL = 128
_TPB = 4
_NB = 2


@functools.lru_cache(maxsize=None)
def _make_gather(T: int):
    tri, tin = T // 8, 8
    rpt = _TPB * _NB
    mesh = plsc.VectorSubcoreMesh(core_axis_name="c", subcore_axis_name="s")

    @functools.partial(
        pl.kernel,
        mesh=mesh,
        out_type=jax.ShapeDtypeStruct((T, _D // 8, _NW, 8, _BL), jnp.float32),
        scratch_types=[
            pltpu.VMEM((tri, tin, _BL), jnp.int32),
            pltpu.VMEM((_NB, _TPB * _BL, _D), jnp.float32),
            pltpu.VMEM((_NB, _TPB, _D // 8, 8, _BL), jnp.float32),
            pltpu.SemaphoreType.DMA((_NB,)),
            pltpu.SemaphoreType.DMA((_NB,)),
        ],
        compiler_params=pltpu.CompilerParams(
            use_tc_tiling_on_sc=False, needs_layout_passes=False
        ),
    )
    def gather(idx5_hbm, table_hbm, out5_hbm, idx_v, rows_v, outt_v, gsem, ssem):
        wid = lax.axis_index("s") * _NC + lax.axis_index("c")
        pltpu.sync_copy(idx5_hbm.at[:, wid], idx_v)

        iota16 = lax.iota(jnp.int32, 16)

        def g_copy(t, b, tt):
            return pltpu.make_async_copy(
                table_hbm.at[idx_v.at[t // 8, t % 8]],
                rows_v.at[b, pl.ds(tt * _BL, _BL)],
                gsem.at[b],
            )

        def s_copy(t, b, tt):
            return pltpu.make_async_copy(
                outt_v.at[b, tt], out5_hbm.at[t, :, wid], ssem.at[b]
            )

        for b in range(_NB):
            for tt in range(_TPB):
                g_copy(b * _TPB + tt, b, tt).start()

        def body(ro, carry):
            for b in range(_NB):
                t0 = (ro * _NB + b) * _TPB
                for tt in range(_TPB):
                    g_copy(t0 + tt, b, tt).wait()

                @pl.when(ro > 0)
                def _():
                    for tt in range(_TPB):
                        s_copy(t0 - rpt + tt, b, tt).wait()

                def tbody(tt, c2):
                    ttv = jnp.full((16,), 0, jnp.int32) + tt
                    blv = [iota16 + (16 * k) + tt * _BL for k in range(8)]
                    blo = [iota16 + (16 * k) for k in range(8)]
                    for d in range(_D):
                        cv = (iota16 + d) & (_D - 1)
                        crv = cv >> 3
                        civ = cv & 7
                        for k in range(8):
                            v = plsc.load_gather(rows_v.at[b], [blv[k], cv])
                            plsc.store_scatter(
                                outt_v.at[b], [ttv, crv, civ, blo[k]], v
                            )
                    s_copy(t0 + tt, b, tt).start()
                    return c2

                lax.fori_loop(0, _TPB, tbody, 0)

                @pl.when(t0 + rpt < T)
                def _():
                    for tt in range(_TPB):
                        g_copy(t0 + rpt + tt, b, tt).start()

            return carry

        lax.fori_loop(0, T // rpt, body, 0)

        for b in range(_NB):
            for tt in range(_TPB):
                s_copy(T - rpt + b * _TPB + tt, b, tt).wait()

    return gather


def kernel(indices, W):
    rows, cols = indices.shape
    idx5 = indices.reshape(_NW, _BL, cols // 8, 8).transpose(2, 0, 3, 1)
    out5 = _make_gather(cols)(idx5, W)
    return out5.transpose(2, 4, 0, 1, 3).reshape(rows, cols, _D)


# speedup vs baseline: 1.6598x; 1.0543x over previous
import functools

import jax
import jax.numpy as jnp
from jax import lax
from jax.experimental import pallas as pl
from jax.experimental.pallas import tpu as pltpu
from jax.experimental.pallas import tpu_sc as plsc

_D = 32
_NC = 2
_NS = 16
_NW = _NC * _NS
_BL = 128
_TPB = 4
_NB = 2


@functools.lru_cache(maxsize=None)
def _make_detrans(nblk: int):
    npt = -(-nblk // _NW)
    ni = -(-npt // 2)
    mesh = plsc.VectorSubcoreMesh(core_axis_name="c", subcore_axis_name="s")

    @functools.partial(
        pl.kernel,
        mesh=mesh,
        out_type=jax.ShapeDtypeStruct((nblk * _BL, _D), jnp.float32),
        scratch_types=[
            pltpu.VMEM((2, _D // 8, 8, _BL), jnp.float32),
            pltpu.VMEM((2, _BL, _D), jnp.float32),
            pltpu.SemaphoreType.DMA((2,)),
            pltpu.SemaphoreType.DMA((2,)),
        ],
        compiler_params=pltpu.CompilerParams(
            use_tc_tiling_on_sc=False, needs_layout_passes=False
        ),
    )
    def detrans(w5_hbm, wlin_hbm, tin_v, tout_v, gsem, ssem):
        wid = lax.axis_index("s") * _NC + lax.axis_index("c")
        iota16 = lax.iota(jnp.int32, 16)

        def blk_of(i, b):
            return wid + _NW * (2 * i + b)

        def g_copy(blk, b):
            return pltpu.make_async_copy(
                w5_hbm.at[:, blk], tin_v.at[b], gsem.at[b]
            )

        def s_copy(blk, b):
            return pltpu.make_async_copy(
                tout_v.at[b], wlin_hbm.at[pl.ds(blk * _BL, _BL)], ssem.at[b]
            )

        for b in range(2):
            g_copy(blk_of(0, b), b).start()

        def body(i, carry):
            for b in range(2):
                blk = blk_of(i, b)

                @pl.when(blk < nblk)
                def _():
                    g_copy(blk, b).wait()

                    @pl.when(i > 0)
                    def _():
                        s_copy(blk, b).wait()

                    for d in range(_D):
                        cv = (iota16 + d) & (_D - 1)
                        crv = cv >> 3
                        civ = cv & 7
                        for k in range(8):
                            blv = iota16 + 16 * k
                            v = plsc.load_gather(tin_v.at[b], [crv, civ, blv])
                            plsc.store_scatter(tout_v.at[b], [blv, cv], v)

                    s_copy(blk, b).start()
                    nxt = blk_of(i + 1, b)

                    @pl.when(nxt < nblk)
                    def _():
                        g_copy(nxt, b).start()

            return carry

        lax.fori_loop(0, ni, body, 0)
        for b in range(2):
            s_copy(blk_of(0, b), b).wait()

    return detrans


@functools.lru_cache(maxsize=None)
def _make_gather(T: int):
    tri, tin = T // 8, 8
    rpt = _TPB * _NB
    mesh = plsc.VectorSubcoreMesh(core_axis_name="c", subcore_axis_name="s")

    @functools.partial(
        pl.kernel,
        mesh=mesh,
        out_type=jax.ShapeDtypeStruct((T, _D // 8, _NW, 8, _BL), jnp.float32),
        scratch_types=[
            pltpu.VMEM((tri, tin, _BL), jnp.int32),
            pltpu.VMEM((_NB, _TPB * _BL, _D), jnp.float32),
            pltpu.VMEM((_NB, _TPB, _D // 8, 8, _BL), jnp.float32),
            pltpu.SemaphoreType.DMA((_NB,)),
            pltpu.SemaphoreType.DMA((_NB,)),
        ],
        compiler_params=pltpu.CompilerParams(
            use_tc_tiling_on_sc=False, needs_layout_passes=False
        ),
    )
    def gather(idx5_hbm, table_hbm, out5_hbm, idx_v, rows_v, outt_v, gsem, ssem):
        wid = lax.axis_index("s") * _NC + lax.axis_index("c")
        pltpu.sync_copy(idx5_hbm.at[:, wid], idx_v)

        iota16 = lax.iota(jnp.int32, 16)

        def g_copy(t, b, tt):
            return pltpu.make_async_copy(
                table_hbm.at[idx_v.at[t // 8, t % 8]],
                rows_v.at[b, pl.ds(tt * _BL, _BL)],
                gsem.at[b],
            )

        def s_copy(t, b, tt):
            return pltpu.make_async_copy(
                outt_v.at[b, tt], out5_hbm.at[t, :, wid], ssem.at[b]
            )

        for b in range(_NB):
            for tt in range(_TPB):
                g_copy(b * _TPB + tt, b, tt).start()

        def body(ro, carry):
            for b in range(_NB):
                t0 = (ro * _NB + b) * _TPB
                for tt in range(_TPB):
                    g_copy(t0 + tt, b, tt).wait()

                @pl.when(ro > 0)
                def _():
                    for tt in range(_TPB):
                        s_copy(t0 - rpt + tt, b, tt).wait()

                def tbody(tt, c2):
                    ttv = jnp.full((16,), 0, jnp.int32) + tt
                    blv = [iota16 + (16 * k) + tt * _BL for k in range(8)]
                    blo = [iota16 + (16 * k) for k in range(8)]
                    for d in range(_D):
                        cv = (iota16 + d) & (_D - 1)
                        crv = cv >> 3
                        civ = cv & 7
                        for k in range(8):
                            v = plsc.load_gather(rows_v.at[b], [blv[k], cv])
                            plsc.store_scatter(
                                outt_v.at[b], [ttv, crv, civ, blo[k]], v
                            )
                    s_copy(t0 + tt, b, tt).start()
                    return c2

                lax.fori_loop(0, _TPB, tbody, 0)

                @pl.when(t0 + rpt < T)
                def _():
                    for tt in range(_TPB):
                        g_copy(t0 + rpt + tt, b, tt).start()

            return carry

        lax.fori_loop(0, T // rpt, body, 0)

        for b in range(_NB):
            for tt in range(_TPB):
                s_copy(T - rpt + b * _TPB + tt, b, tt).wait()

    return gather


def kernel(indices, W):
    rows, cols = indices.shape
    vocab = W.shape[0]
    nblk = -(-vocab // _BL)
    w_pad = jnp.pad(W, ((0, nblk * _BL - vocab), (0, 0)))
    w5 = w_pad.T.reshape(_D // 8, 8, nblk, _BL).transpose(0, 2, 1, 3)
    wlin = _make_detrans(nblk)(w5)
    idx5 = indices.reshape(_NW, _BL, cols // 8, 8).transpose(2, 0, 3, 1)
    out5 = _make_gather(cols)(idx5, wlin)
    return out5.transpose(2, 4, 0, 1, 3).reshape(rows, cols, _D)


# speedup vs baseline: 1.7706x; 1.0667x over previous
import functools

import jax
import jax.numpy as jnp
from jax import lax
from jax.experimental import pallas as pl
from jax.experimental.pallas import tpu as pltpu
from jax.experimental.pallas import tpu_sc as plsc

_D = 32
_P = 32
_NC = 2
_NS = 16
_NW = _NC * _NS
_BL = 128
_TPB = 4
_NB = 2


@functools.lru_cache(maxsize=None)
def _make_detrans(nblk: int):
    npt = -(-nblk // _NW)
    ni = -(-npt // 2)
    mesh = plsc.VectorSubcoreMesh(core_axis_name="c", subcore_axis_name="s")

    @functools.partial(
        pl.kernel,
        mesh=mesh,
        out_type=jax.ShapeDtypeStruct((nblk, _BL * _P), jnp.float32),
        scratch_types=[
            pltpu.VMEM((2, _D, _BL), jnp.float32),
            pltpu.VMEM((2, _BL * _P), jnp.float32),
            pltpu.SemaphoreType.DMA((2,)),
            pltpu.SemaphoreType.DMA((2,)),
        ],
        compiler_params=pltpu.CompilerParams(
            use_tc_tiling_on_sc=False, needs_layout_passes=False
        ),
    )
    def detrans(w5_hbm, wlin_hbm, tin_v, tout_v, gsem, ssem):
        wid = lax.axis_index("s") * _NC + lax.axis_index("c")
        iota16 = lax.iota(jnp.int32, 16)
        blp = [iota16 * _P + 16 * _P * k for k in range(8)]

        def blk_of(i, b):
            return wid + _NW * (2 * i + b)

        def g_copies(blk, b):
            return [
                pltpu.make_async_copy(
                    w5_hbm.at[cr, blk],
                    tin_v.at[b, pl.ds(8 * cr, 8)],
                    gsem.at[b],
                )
                for cr in range(_D // 8)
            ]

        def s_copy(blk, b):
            return pltpu.make_async_copy(
                tout_v.at[b], wlin_hbm.at[blk], ssem.at[b]
            )

        for b in range(2):
            for c in g_copies(blk_of(0, b), b):
                c.start()

        def body(i, carry):
            for b in range(2):
                blk = blk_of(i, b)

                @pl.when(blk < nblk)
                def _():
                    for c in g_copies(blk, b):
                        c.wait()

                    @pl.when(i > 0)
                    def _():
                        s_copy(blk, b).wait()

                    for d in range(_D):
                        cv = (iota16 + d) & (_D - 1)
                        for k in range(8):
                            v = plsc.load_gather(
                                tin_v.at[b], [cv, iota16 + 16 * k]
                            )
                            plsc.store_scatter(
                                tout_v.at[b], [blp[k] + cv], v
                            )

                    s_copy(blk, b).start()
                    nxt = blk_of(i + 1, b)

                    @pl.when(nxt < nblk)
                    def _():
                        for c in g_copies(nxt, b):
                            c.start()

            return carry

        lax.fori_loop(0, ni, body, 0)
        for b in range(2):
            s_copy(blk_of(0, b), b).wait()

    return detrans


@functools.lru_cache(maxsize=None)
def _make_gather(T: int):
    tri, tin = T // 8, 8
    rpt = _TPB * _NB
    mesh = plsc.VectorSubcoreMesh(core_axis_name="c", subcore_axis_name="s")

    @functools.partial(
        pl.kernel,
        mesh=mesh,
        out_type=jax.ShapeDtypeStruct((T, _D // 8, _NW, 8, _BL), jnp.float32),
        scratch_types=[
            pltpu.VMEM((tri, tin, _BL), jnp.int32),
            pltpu.VMEM((_NB, _TPB * _BL, _P), jnp.float32),
            pltpu.VMEM((_NB, _TPB, _D // 8, 8, _BL), jnp.float32),
            pltpu.SemaphoreType.DMA((_NB,)),
            pltpu.SemaphoreType.DMA((_NB,)),
        ],
        compiler_params=pltpu.CompilerParams(
            use_tc_tiling_on_sc=False, needs_layout_passes=False
        ),
    )
    def gather(idx5_hbm, table_hbm, out5_hbm, idx_v, rows_v, outt_v, gsem, ssem):
        wid = lax.axis_index("s") * _NC + lax.axis_index("c")
        pltpu.sync_copy(idx5_hbm.at[:, wid], idx_v)

        iota16 = lax.iota(jnp.int32, 16)

        def g_copy(t, b, tt):
            return pltpu.make_async_copy(
                table_hbm.at[idx_v.at[t // 8, t % 8]],
                rows_v.at[b, pl.ds(tt * _BL, _BL)],
                gsem.at[b],
            )

        def s_copy(t, b, tt):
            return pltpu.make_async_copy(
                outt_v.at[b, tt], out5_hbm.at[t, :, wid], ssem.at[b]
            )

        for b in range(_NB):
            for tt in range(_TPB):
                g_copy(b * _TPB + tt, b, tt).start()

        def body(ro, carry):
            for b in range(_NB):
                t0 = (ro * _NB + b) * _TPB
                for tt in range(_TPB):
                    g_copy(t0 + tt, b, tt).wait()

                @pl.when(ro > 0)
                def _():
                    for tt in range(_TPB):
                        s_copy(t0 - rpt + tt, b, tt).wait()

                def tbody(tt, c2):
                    rv = [iota16 + (tt * _BL + 16 * m) for m in range(8)]
                    blo = [iota16 + 16 * m for m in range(8)]
                    for d in range(_D):
                        cv = (iota16 + d) & (_D - 1)
                        crv = cv >> 3
                        civ = cv & 7
                        for m in range(8):
                            v = plsc.load_gather(rows_v.at[b], [rv[m], cv])
                            plsc.store_scatter(
                                outt_v.at[b, tt], [crv, civ, blo[m]], v
                            )
                    s_copy(t0 + tt, b, tt).start()
                    return c2

                lax.fori_loop(0, _TPB, tbody, 0)

                @pl.when(t0 + rpt < T)
                def _():
                    for tt in range(_TPB):
                        g_copy(t0 + rpt + tt, b, tt).start()

            return carry

        lax.fori_loop(0, T // rpt, body, 0)

        for b in range(_NB):
            for tt in range(_TPB):
                s_copy(T - rpt + b * _TPB + tt, b, tt).wait()

    return gather


def kernel(indices, W):
    rows, cols = indices.shape
    vocab = W.shape[0]
    nblk = -(-vocab // _BL)
    w_pad = jnp.pad(W, ((0, nblk * _BL - vocab), (0, 0)))
    w5 = w_pad.T.reshape(_D // 8, 8, nblk, _BL).transpose(0, 2, 1, 3)
    wlin = _make_detrans(nblk)(w5).reshape(nblk * _BL, _P)
    idx5 = indices.reshape(_NW, _BL, cols // 8, 8).transpose(2, 0, 3, 1)
    out5 = _make_gather(cols)(idx5, wlin)
    return out5.transpose(2, 4, 0, 1, 3).reshape(rows, cols, _D)


# speedup vs baseline: 1.7782x; 1.0043x over previous
import functools

import jax
import jax.numpy as jnp
from jax import lax
from jax.experimental import pallas as pl
from jax.experimental.pallas import tpu as pltpu
from jax.experimental.pallas import tpu_sc as plsc

_D = 32
_P = 32
_NC = 2
_NS = 16
_NW = _NC * _NS
_BL = 128
_TPB = 5
_NB = 2


@functools.lru_cache(maxsize=None)
def _make_detrans(nblk: int):
    npt = -(-nblk // _NW)
    ni = -(-npt // 2)
    mesh = plsc.VectorSubcoreMesh(core_axis_name="c", subcore_axis_name="s")

    @functools.partial(
        pl.kernel,
        mesh=mesh,
        out_type=jax.ShapeDtypeStruct((nblk, _BL * _P), jnp.float32),
        scratch_types=[
            pltpu.VMEM((2, _D, _BL), jnp.float32),
            pltpu.VMEM((2, _BL * _P), jnp.float32),
            pltpu.SemaphoreType.DMA((2,)),
            pltpu.SemaphoreType.DMA((2,)),
        ],
        compiler_params=pltpu.CompilerParams(
            use_tc_tiling_on_sc=False, needs_layout_passes=False
        ),
    )
    def detrans(w5_hbm, wlin_hbm, tin_v, tout_v, gsem, ssem):
        wid = lax.axis_index("s") * _NC + lax.axis_index("c")
        iota16 = lax.iota(jnp.int32, 16)
        blp = [iota16 * _P + 16 * _P * k for k in range(8)]

        def blk_of(i, b):
            return wid + _NW * (2 * i + b)

        def g_copies(blk, b):
            return [
                pltpu.make_async_copy(
                    w5_hbm.at[cr, blk],
                    tin_v.at[b, pl.ds(8 * cr, 8)],
                    gsem.at[b],
                )
                for cr in range(_D // 8)
            ]

        def s_copy(blk, b):
            return pltpu.make_async_copy(
                tout_v.at[b], wlin_hbm.at[blk], ssem.at[b]
            )

        for b in range(2):
            for c in g_copies(blk_of(0, b), b):
                c.start()

        def body(i, carry):
            for b in range(2):
                blk = blk_of(i, b)

                @pl.when(blk < nblk)
                def _():
                    for c in g_copies(blk, b):
                        c.wait()

                    @pl.when(i > 0)
                    def _():
                        s_copy(blk, b).wait()

                    for d in range(_D):
                        cv = (iota16 + d) & (_D - 1)
                        for k in range(8):
                            v = plsc.load_gather(
                                tin_v.at[b], [cv, iota16 + 16 * k]
                            )
                            plsc.store_scatter(
                                tout_v.at[b], [blp[k] + cv], v
                            )

                    s_copy(blk, b).start()
                    nxt = blk_of(i + 1, b)

                    @pl.when(nxt < nblk)
                    def _():
                        for c in g_copies(nxt, b):
                            c.start()

            return carry

        lax.fori_loop(0, ni, body, 0)
        for b in range(2):
            s_copy(blk_of(0, b), b).wait()

    return detrans


@functools.lru_cache(maxsize=None)
def _make_gather(T: int):
    tri, tin = T // 8, 8
    rpt = _TPB * _NB
    mesh = plsc.VectorSubcoreMesh(core_axis_name="c", subcore_axis_name="s")

    @functools.partial(
        pl.kernel,
        mesh=mesh,
        out_type=jax.ShapeDtypeStruct((T, _D // 8, _NW, 8, _BL), jnp.float32),
        scratch_types=[
            pltpu.VMEM((tri, tin, _BL), jnp.int32),
            pltpu.VMEM((_NB, _TPB * _BL, _P), jnp.float32),
            pltpu.VMEM((_NB, _TPB, _D // 8, 8, _BL), jnp.float32),
            pltpu.SemaphoreType.DMA((_NB,)),
            pltpu.SemaphoreType.DMA((_NB,)),
        ],
        compiler_params=pltpu.CompilerParams(
            use_tc_tiling_on_sc=False, needs_layout_passes=False
        ),
    )
    def gather(idx5_hbm, table_hbm, out5_hbm, idx_v, rows_v, outt_v, gsem, ssem):
        wid = lax.axis_index("s") * _NC + lax.axis_index("c")
        pltpu.sync_copy(idx5_hbm.at[:, wid], idx_v)

        iota16 = lax.iota(jnp.int32, 16)

        def g_copy(t, b, tt):
            return pltpu.make_async_copy(
                table_hbm.at[idx_v.at[t // 8, t % 8]],
                rows_v.at[b, pl.ds(tt * _BL, _BL)],
                gsem.at[b],
            )

        def s_copy(t, b, tt):
            return pltpu.make_async_copy(
                outt_v.at[b, tt], out5_hbm.at[t, :, wid], ssem.at[b]
            )

        for b in range(_NB):
            for tt in range(_TPB):
                g_copy(b * _TPB + tt, b, tt).start()

        def body(ro, carry):
            for b in range(_NB):
                t0 = (ro * _NB + b) * _TPB
                for tt in range(_TPB):
                    g_copy(t0 + tt, b, tt).wait()

                @pl.when(ro > 0)
                def _():
                    for tt in range(_TPB):
                        s_copy(t0 - rpt + tt, b, tt).wait()

                def tbody(tt, c2):
                    rv = [iota16 + (tt * _BL + 16 * m) for m in range(8)]
                    blo = [iota16 + 16 * m for m in range(8)]
                    for d in range(_D):
                        cv = (iota16 + d) & (_D - 1)
                        crv = cv >> 3
                        civ = cv & 7
                        for m in range(8):
                            v = plsc.load_gather(rows_v.at[b], [rv[m], cv])
                            plsc.store_scatter(
                                outt_v.at[b, tt], [crv, civ, blo[m]], v
                            )
                    s_copy(t0 + tt, b, tt).start()
                    return c2

                lax.fori_loop(0, _TPB, tbody, 0)

                @pl.when(t0 + rpt < T)
                def _():
                    for tt in range(_TPB):
                        g_copy(t0 + rpt + tt, b, tt).start()

            return carry

        lax.fori_loop(0, T // rpt, body, 0)

        for b in range(_NB):
            for tt in range(_TPB):
                s_copy(T - rpt + b * _TPB + tt, b, tt).wait()

    return gather


def kernel(indices, W):
    rows, cols = indices.shape
    vocab = W.shape[0]
    nblk = -(-vocab // _BL)
    w_pad = jnp.pad(W, ((0, nblk * _BL - vocab), (0, 0)))
    w5 = w_pad.T.reshape(_D // 8, 8, nblk, _BL).transpose(0, 2, 1, 3)
    wlin = _make_detrans(nblk)(w5).reshape(nblk * _BL, _P)
    idx5 = indices.reshape(_NW, _BL, cols // 8, 8).transpose(2, 0, 3, 1)
    out5 = _make_gather(cols)(idx5, wlin)
    return out5.transpose(2, 4, 0, 1, 3).reshape(rows, cols, _D)
